# Initial kernel scaffold; baseline (speedup 1.0000x reference)
import jax
import jax.numpy as jnp
from jax.experimental import pallas as pl


def kernel(species, edge_src, edge_dst, distances, switch, species_table, Wself0, Wself1, bself, mW1_0, mW1_1, mb1, mW2, mb2, oW1, ob1, oW2, ob2):
    raise NotImplementedError("write your pallas kernel here")


# speedup vs baseline: 1.2968x; 1.2968x over previous
import functools

import jax
import jax.numpy as jnp
from jax import lax
from jax.experimental import pallas as pl
from jax.experimental.pallas import tpu as pltpu
from jax.experimental.pallas import tpu_sc as plsc

N = 50000
E = 800000
DIM = 80
RDIM = 20
SDIM = 16
NSPEC = 50
CUTOFF = 5.0
NLAYERS = 2
NONSITE = 3
ETA = (RDIM / (1.0 - 1.0 / CUTOFF)) ** 2

HALF = N // 2
NTILE = 16
ROWS_PER_TILE = 1563
HPAD = NTILE * ROWS_PER_TILE
DUMMY = HPAD - 1

CH = 128
CHUNKS = E // CH
NWORK = 32
GATHER_TRIPS_BASE = CHUNKS // NWORK
GATHER_EXTRA = CHUNKS % NWORK
CHS = 64
SCHUNKS = E // CHS
SCAT_TRIPS_BASE = SCHUNKS // NTILE
SCAT_EXTRA = SCHUNKS % NTILE

BN = 1000
GN = N // BN
NBH = GN // 2
BE = 6400
GE = E // BE

_F32 = jnp.float32


def _silu(x):
    return x * (1.0 / (1.0 + jnp.exp(-x)))


def _node_pre0(species2, table, w_u, w_self, b_self):
    def body(sp_ref, tab_ref, wu_ref, ws_ref, bs_ref, u_ref, zs_ref):
        sp = sp_ref[...]
        iota = lax.broadcasted_iota(jnp.int32, (BN, NSPEC), 1)
        oh = (iota == sp).astype(_F32)
        zi = jnp.dot(oh, tab_ref[...], preferred_element_type=_F32)
        u_ref[...] = jnp.dot(zi, wu_ref[...], preferred_element_type=_F32)
        zs = jnp.dot(zi, ws_ref[...], preferred_element_type=_F32) + bs_ref[...]
        zs_ref[...] = zs[None]

    return pl.pallas_call(
        body,
        grid=(GN,),
        in_specs=[
            pl.BlockSpec((BN, 1), lambda j: (j, 0)),
            pl.BlockSpec((NSPEC, SDIM), lambda j: (0, 0)),
            pl.BlockSpec((SDIM, DIM), lambda j: (0, 0)),
            pl.BlockSpec((SDIM, DIM), lambda j: (0, 0)),
            pl.BlockSpec((1, DIM), lambda j: (0, 0)),
        ],
        out_specs=[
            pl.BlockSpec((BN, DIM), lambda j: (j, 0)),
            pl.BlockSpec((1, BN, DIM), lambda j: (j // NBH, j % NBH, 0)),
        ],
        out_shape=[
            jax.ShapeDtypeStruct((N, DIM), _F32),
            jax.ShapeDtypeStruct((2, HPAD, DIM), _F32),
        ],
    )(species2, table, w_u, w_self, b_self)


def _sc_gather(u, edge_dst):
    mesh = plsc.VectorSubcoreMesh(core_axis_name="c", subcore_axis_name="s")

    @functools.partial(
        pl.kernel,
        out_type=jax.ShapeDtypeStruct((E, DIM), _F32),
        mesh=mesh,
        compiler_params=pltpu.CompilerParams(use_tc_tiling_on_sc=False),
        scratch_types=[
            pltpu.VMEM((CH,), jnp.int32),
            pltpu.VMEM((CH, DIM), _F32),
            pltpu.SemaphoreType.DMA,
        ],
    )
    def k(u_hbm, dst_hbm, g_hbm, idx_v, rows_v, sem):
        w = lax.axis_index("s") * 2 + lax.axis_index("c")
        trips = GATHER_TRIPS_BASE + (w < GATHER_EXTRA).astype(jnp.int32)

        def body(i, carry):
            ch = w + NWORK * i
            off = ch * CH
            pltpu.sync_copy(dst_hbm.at[pl.ds(off, CH)], idx_v)
            pltpu.async_copy(u_hbm.at[idx_v], rows_v, sem).wait()
            pltpu.sync_copy(rows_v, g_hbm.at[pl.ds(off, CH)])
            return carry

        lax.fori_loop(0, trips, body, 0)

    return k(u, edge_dst)


def _edge_mlp(g, dist2, sw2, mus2, w_s, b1, w2, b2):
    def body(g_ref, d_ref, sw_ref, mus_ref, ws_ref, b1_ref, w2_ref, b2_ref,
             m_ref):
        rinv = 1.0 / d_ref[...]
        s = jnp.exp(-ETA * (rinv - mus_ref[...]) ** 2)
        pre = (g_ref[...]
               + jnp.dot(s, ws_ref[...], preferred_element_type=_F32)
               + b1_ref[...])
        h = _silu(pre)
        mm = jnp.dot(h, w2_ref[...], preferred_element_type=_F32) + b2_ref[...]
        m_ref[...] = mm * sw_ref[...]

    return pl.pallas_call(
        body,
        grid=(GE,),
        in_specs=[
            pl.BlockSpec((BE, DIM), lambda j: (j, 0)),
            pl.BlockSpec((BE, 1), lambda j: (j, 0)),
            pl.BlockSpec((BE, 1), lambda j: (j, 0)),
            pl.BlockSpec((1, RDIM), lambda j: (0, 0)),
            pl.BlockSpec((RDIM, DIM), lambda j: (0, 0)),
            pl.BlockSpec((1, DIM), lambda j: (0, 0)),
            pl.BlockSpec((DIM, DIM), lambda j: (0, 0)),
            pl.BlockSpec((1, DIM), lambda j: (0, 0)),
        ],
        out_specs=pl.BlockSpec((BE, DIM), lambda j: (j, 0)),
        out_shape=jax.ShapeDtypeStruct((E, DIM), _F32),
    )(g, dist2, sw2, mus2, w_s, b1, w2, b2)


def _sc_scatter(m, edge_src, zself_pad):
    mesh = plsc.VectorSubcoreMesh(core_axis_name="c", subcore_axis_name="s")

    @functools.partial(
        pl.kernel,
        out_type=jax.ShapeDtypeStruct((2, HPAD, DIM), _F32),
        mesh=mesh,
        compiler_params=pltpu.CompilerParams(use_tc_tiling_on_sc=False),
        scratch_types=[
            pltpu.VMEM((CHS,), jnp.int32),
            pltpu.VMEM((CHS,), jnp.int32),
            pltpu.VMEM((CHS, DIM), _F32),
            pltpu.VMEM_SHARED((HPAD, DIM), _F32),
            pltpu.SemaphoreType.DMA,
        ],
    )
    def k(m_hbm, src_hbm, zs_hbm, out_hbm, srcv, idxv, rows_v, acc, sem):
        c = lax.axis_index("c")
        s = lax.axis_index("s")
        base = c * HALF
        r0 = s * ROWS_PER_TILE
        pltpu.sync_copy(zs_hbm.at[c, pl.ds(r0, ROWS_PER_TILE)],
                        acc.at[pl.ds(r0, ROWS_PER_TILE)])
        plsc.subcore_barrier()

        trips = SCAT_TRIPS_BASE + (s < SCAT_EXTRA).astype(jnp.int32)

        def body(i, carry):
            ch = s + NTILE * i
            off = ch * CHS
            pltpu.sync_copy(src_hbm.at[pl.ds(off, CHS)], srcv)
            pltpu.sync_copy(m_hbm.at[pl.ds(off, CHS)], rows_v)
            for v in range(CHS // 16):
                sl = pl.ds(v * 16, 16)
                loc = srcv[sl] - base
                ok = (loc >= 0) & (loc < HALF)
                idxv[sl] = jnp.where(ok, loc, DUMMY)
            pltpu.async_copy(rows_v, acc.at[idxv], sem, add=True).wait()
            return carry

        lax.fori_loop(0, trips, body, 0)
        plsc.subcore_barrier()
        pltpu.sync_copy(acc.at[pl.ds(r0, ROWS_PER_TILE)],
                        out_hbm.at[c, pl.ds(r0, ROWS_PER_TILE)])

    return k(m, edge_src, zself_pad)


def _node_post(accout, ow1, ob1, ow2, ob2, nxt):
    has_next = nxt is not None

    def body(acc_ref, w1_ref, b1_ref, w2_ref, b2_ref, *rest):
        if has_next:
            wu_ref, ws_ref, bs_ref, z_ref, u_ref, zs_ref = rest
        else:
            (z_ref,) = rest
        x = _silu(acc_ref[0])
        for j in range(NONSITE):
            hh = _silu(jnp.dot(x, w1_ref[j], preferred_element_type=_F32)
                       + b1_ref[j])
            x = x + jnp.dot(hh, w2_ref[j], preferred_element_type=_F32) + b2_ref[j]
        z_ref[...] = x
        if has_next:
            u_ref[...] = jnp.dot(x, wu_ref[...], preferred_element_type=_F32)
            zs = jnp.dot(x, ws_ref[...], preferred_element_type=_F32) + bs_ref[...]
            zs_ref[...] = zs[None]

    in_specs = [
        pl.BlockSpec((1, BN, DIM), lambda j: (j // NBH, j % NBH, 0)),
        pl.BlockSpec((NONSITE, DIM, DIM), lambda j: (0, 0, 0)),
        pl.BlockSpec((NONSITE, 1, DIM), lambda j: (0, 0, 0)),
        pl.BlockSpec((NONSITE, DIM, DIM), lambda j: (0, 0, 0)),
        pl.BlockSpec((NONSITE, 1, DIM), lambda j: (0, 0, 0)),
    ]
    out_specs = [pl.BlockSpec((BN, DIM), lambda j: (j, 0))]
    out_shape = [jax.ShapeDtypeStruct((N, DIM), _F32)]
    args = [accout, ow1, ob1, ow2, ob2]
    if has_next:
        w_u, w_self, b_self = nxt
        in_specs += [
            pl.BlockSpec((DIM, DIM), lambda j: (0, 0)),
            pl.BlockSpec((DIM, DIM), lambda j: (0, 0)),
            pl.BlockSpec((1, DIM), lambda j: (0, 0)),
        ]
        out_specs += [
            pl.BlockSpec((BN, DIM), lambda j: (j, 0)),
            pl.BlockSpec((1, BN, DIM), lambda j: (j // NBH, j % NBH, 0)),
        ]
        out_shape += [
            jax.ShapeDtypeStruct((N, DIM), _F32),
            jax.ShapeDtypeStruct((2, HPAD, DIM), _F32),
        ]
        args += [w_u, w_self, b_self]

    return pl.pallas_call(
        body,
        grid=(GN,),
        in_specs=in_specs,
        out_specs=out_specs,
        out_shape=out_shape,
    )(*args)


def kernel(species, edge_src, edge_dst, distances, switch, species_table,
           Wself0, Wself1, bself, mW1_0, mW1_1, mb1, mW2, mb2,
           oW1, ob1, oW2, ob2):
    species2 = species.reshape(N, 1).astype(jnp.int32)
    dist2 = distances.reshape(E, 1)
    sw2 = switch.reshape(E, 1)
    mus2 = jnp.linspace(1.0 / CUTOFF, 1.0, RDIM).reshape(1, RDIM)
    edge_src = edge_src.astype(jnp.int32)
    edge_dst = edge_dst.astype(jnp.int32)

    ob1r = ob1.reshape(NLAYERS, NONSITE, 1, DIM)
    ob2r = ob2.reshape(NLAYERS, NONSITE, 1, DIM)

    u, zself = _node_pre0(species2, species_table, mW1_0[:SDIM], Wself0,
                          bself[0].reshape(1, DIM))
    g = _sc_gather(u, edge_dst)
    m = _edge_mlp(g, dist2, sw2, mus2, mW1_0[SDIM:],
                  mb1[0].reshape(1, DIM), mW2[0], mb2[0].reshape(1, DIM))
    acc = _sc_scatter(m, edge_src, zself)
    z0, u, zself = _node_post(acc, oW1[0], ob1r[0], oW2[0], ob2r[0],
                              (mW1_1[:DIM], Wself1, bself[1].reshape(1, DIM)))

    g = _sc_gather(u, edge_dst)
    m = _edge_mlp(g, dist2, sw2, mus2, mW1_1[DIM:],
                  mb1[1].reshape(1, DIM), mW2[1], mb2[1].reshape(1, DIM))
    acc = _sc_scatter(m, edge_src, zself)
    (z1,) = _node_post(acc, oW1[1], ob1r[1], oW2[1], ob2r[1], None)

    return jnp.stack([z0, z1], axis=1)


# speedup vs baseline: 1.4353x; 1.1068x over previous
import functools

import jax
import jax.numpy as jnp
from jax import lax
from jax.experimental import pallas as pl
from jax.experimental.pallas import tpu as pltpu
from jax.experimental.pallas import tpu_sc as plsc

N = 50000
E = 800000
DIM = 80
HD = DIM // 2
RDIM = 20
SDIM = 16
NSPEC = 50
CUTOFF = 5.0
NLAYERS = 2
NONSITE = 3
ETA = (RDIM / (1.0 - 1.0 / CUTOFF)) ** 2

NTILE = 16
NWORK = 32

CH = 128
E_PAD = 802816
GCH_W = E_PAD // CH // NWORK
GGRP_W = GCH_W // 4
SCH_T = E_PAD // CH // NTILE
SGRP_T = SCH_T // 4

PHROWS = 25088
ROWS_PER_TILE = PHROWS // NTILE
NPAD2 = 2 * PHROWS
DUMMY = PHROWS

BN = 1000
GN = N // BN
BE = 6272
GE = E_PAD // BE

_F32 = jnp.float32


def _silu(x):
    return x * (1.0 / (1.0 + jnp.exp(-x)))


def _node_pre0(species2, table, w_u, w_self, b_self):
    def body(sp_ref, tab_ref, wu_ref, ws_ref, bs_ref, u_ref, zs_ref):
        sp = sp_ref[...]
        iota = lax.broadcasted_iota(jnp.int32, (BN, NSPEC), 1)
        oh = (iota == sp).astype(_F32)
        zi = jnp.dot(oh, tab_ref[...], preferred_element_type=_F32)
        u_ref[...] = jnp.dot(zi, wu_ref[...], preferred_element_type=_F32)
        zs_ref[...] = (jnp.dot(zi, ws_ref[...], preferred_element_type=_F32)
                       + bs_ref[...])

    return pl.pallas_call(
        body,
        grid=(GN,),
        in_specs=[
            pl.BlockSpec((BN, 1), lambda j: (j, 0)),
            pl.BlockSpec((NSPEC, SDIM), lambda j: (0, 0)),
            pl.BlockSpec((SDIM, DIM), lambda j: (0, 0)),
            pl.BlockSpec((SDIM, DIM), lambda j: (0, 0)),
            pl.BlockSpec((1, DIM), lambda j: (0, 0)),
        ],
        out_specs=[
            pl.BlockSpec((BN, DIM), lambda j: (j, 0)),
            pl.BlockSpec((BN, DIM), lambda j: (j, 0)),
        ],
        out_shape=[
            jax.ShapeDtypeStruct((N, DIM), _F32),
            jax.ShapeDtypeStruct((NPAD2, DIM), _F32),
        ],
    )(species2, table, w_u, w_self, b_self)


def _sc_gather(u, edge_dst):
    mesh = plsc.VectorSubcoreMesh(core_axis_name="c", subcore_axis_name="s")

    scratch = ([pltpu.VMEM((CH,), jnp.int32) for _ in range(8)]
               + [pltpu.VMEM((CH, DIM), _F32) for _ in range(8)]
               + [pltpu.SemaphoreType.DMA] * 3)

    @functools.partial(
        pl.kernel,
        out_type=jax.ShapeDtypeStruct((E_PAD, DIM), _F32),
        mesh=mesh,
        compiler_params=pltpu.CompilerParams(use_tc_tiling_on_sc=False),
        scratch_types=scratch,
    )
    def k(u_hbm, dst_hbm, g_hbm, *sc):
        idxs = sc[0:8]
        rows = sc[8:16]
        sem_i, sem_g, sem_o = sc[16:19]
        w = lax.axis_index("s") * 2 + lax.axis_index("c")
        base = w * (GCH_W * CH)

        def _drain_out(lo):
            for j in range(4):
                pltpu.make_async_copy(rows[(lo ^ 4) + j].at[...],
                                      g_hbm.at[pl.ds(base, CH)],
                                      sem_o).wait()

        def one(g_idx, lo, first):
            off0 = base + g_idx * (4 * CH)
            for j in range(4):
                pltpu.async_copy(dst_hbm.at[pl.ds(off0 + j * CH, CH)],
                                 idxs[lo + j].at[...], sem_i)
            for j in range(4):
                pltpu.make_async_copy(dst_hbm.at[pl.ds(base, CH)],
                                      idxs[lo + j].at[...], sem_i).wait()
            for j in range(4):
                pltpu.async_copy(u_hbm.at[idxs[lo + j]],
                                 rows[lo + j].at[...], sem_g)
            if first is None:
                _drain_out(lo)
            else:
                @pl.when(first)
                def _():
                    _drain_out(lo)
            for j in range(4):
                pltpu.make_async_copy(u_hbm.at[pl.ds(base, CH)],
                                      rows[lo + j].at[...], sem_g).wait()
            for j in range(4):
                pltpu.async_copy(rows[lo + j].at[...],
                                 g_hbm.at[pl.ds(off0 + j * CH, CH)], sem_o)

        def grp2(i, carry):
            g0 = i * 2
            one(g0, 0, g0 > 0)
            one(g0 + 1, 4, None)
            return carry

        lax.fori_loop(0, GGRP_W // 2, grp2, 0)
        one(GGRP_W - 1, 0, None)
        for j in range(4):
            pltpu.make_async_copy(rows[j].at[...],
                                  g_hbm.at[pl.ds(base, CH)], sem_o).wait()

    return k(u, edge_dst)


def _edge_mlp(g, dist2, sw2, mus2, w_s, b1, w2, b2):
    def body(g_ref, d_ref, sw_ref, mus_ref, ws_ref, b1_ref, w2_ref, b2_ref,
             m_ref):
        rinv = 1.0 / d_ref[...]
        s = jnp.exp(-ETA * (rinv - mus_ref[...]) ** 2)
        pre = (g_ref[...]
               + jnp.dot(s, ws_ref[...], preferred_element_type=_F32)
               + b1_ref[...])
        h = _silu(pre)
        mm = jnp.dot(h, w2_ref[...], preferred_element_type=_F32) + b2_ref[...]
        m_ref[...] = mm * sw_ref[...]

    return pl.pallas_call(
        body,
        grid=(GE,),
        in_specs=[
            pl.BlockSpec((BE, DIM), lambda j: (j, 0)),
            pl.BlockSpec((BE, 1), lambda j: (j, 0)),
            pl.BlockSpec((BE, 1), lambda j: (j, 0)),
            pl.BlockSpec((1, RDIM), lambda j: (0, 0)),
            pl.BlockSpec((RDIM, DIM), lambda j: (0, 0)),
            pl.BlockSpec((1, DIM), lambda j: (0, 0)),
            pl.BlockSpec((DIM, DIM), lambda j: (0, 0)),
            pl.BlockSpec((1, DIM), lambda j: (0, 0)),
        ],
        out_specs=pl.BlockSpec((BE, DIM), lambda j: (j, 0)),
        out_shape=jax.ShapeDtypeStruct((E_PAD, DIM), _F32),
    )(g, dist2, sw2, mus2, w_s, b1, w2, b2)


def _sc_scatter(m, edge_src, zself_pad):
    mesh = plsc.VectorSubcoreMesh(core_axis_name="c", subcore_axis_name="s")

    scratch = ([pltpu.VMEM((CH,), jnp.int32) for _ in range(8)]
               + [pltpu.VMEM((CH, HD), _F32) for _ in range(8)]
               + [pltpu.VMEM_SHARED((PHROWS + 8, HD), _F32)]
               + [pltpu.SemaphoreType.DMA] * 2)

    @functools.partial(
        pl.kernel,
        out_type=jax.ShapeDtypeStruct((NPAD2, DIM), _F32),
        mesh=mesh,
        compiler_params=pltpu.CompilerParams(use_tc_tiling_on_sc=False),
        scratch_types=scratch,
    )
    def k(m_hbm, src_hbm, zs_hbm, out_hbm, *sc):
        idxs = sc[0:8]
        rows = sc[8:16]
        acc = sc[16]
        sem_in, sem_a = sc[17:19]
        c = lax.axis_index("c")
        s = lax.axis_index("s")
        r0 = s * ROWS_PER_TILE
        col0 = c * HD
        tbase = s * (SCH_T * CH)

        for phase in range(2):
            prow = phase * PHROWS
            pltpu.sync_copy(
                zs_hbm.at[pl.ds(prow + r0, ROWS_PER_TILE),
                          pl.ds(col0, HD)],
                acc.at[pl.ds(r0, ROWS_PER_TILE)])
            plsc.subcore_barrier()

            def grp(g_i, carry):
                def one(g_idx, lo):
                    off0 = tbase + g_idx * (4 * CH)
                    for j in range(4):
                        pltpu.async_copy(
                            src_hbm.at[pl.ds(off0 + j * CH, CH)],
                            idxs[lo + j].at[...], sem_in)
                        pltpu.async_copy(
                            m_hbm.at[pl.ds(off0 + j * CH, CH),
                                     pl.ds(col0, HD)],
                            rows[lo + j].at[...], sem_in)

                    @pl.when(g_idx > 0)
                    def _():
                        for j in range(4):
                            pltpu.make_async_copy(
                                rows[(lo ^ 4) + j].at[...],
                                acc.at[pl.ds(0, CH)], sem_a).wait()
                    for j in range(4):
                        pltpu.make_async_copy(
                            src_hbm.at[pl.ds(tbase, CH)],
                            idxs[lo + j].at[...], sem_in).wait()
                        pltpu.make_async_copy(
                            m_hbm.at[pl.ds(tbase, CH), pl.ds(col0, HD)],
                            rows[lo + j].at[...], sem_in).wait()
                    for j in range(4):
                        for v in range(CH // 16):
                            sl = pl.ds(v * 16, 16)
                            loc = idxs[lo + j][sl] - prow
                            ok = (loc >= 0) & (loc < PHROWS)
                            idxs[lo + j][sl] = jnp.where(ok, loc, DUMMY)
                    for j in range(4):
                        pltpu.async_copy(rows[lo + j].at[...],
                                         acc.at[idxs[lo + j]], sem_a,
                                         add=True)

                g0 = g_i * 2
                one(g0, 0)
                one(g0 + 1, 4)
                return carry

            lax.fori_loop(0, SGRP_T // 2, grp, 0)
            for j in range(4):
                pltpu.make_async_copy(rows[4 + j].at[...],
                                      acc.at[pl.ds(0, CH)], sem_a).wait()
            plsc.subcore_barrier()
            pltpu.sync_copy(
                acc.at[pl.ds(r0, ROWS_PER_TILE)],
                out_hbm.at[pl.ds(prow + r0, ROWS_PER_TILE),
                           pl.ds(col0, HD)])
            plsc.subcore_barrier()

    return k(m, edge_src, zself_pad)


def _node_post(accout, ow1, ob1, ow2, ob2, nxt):
    has_next = nxt is not None

    def body(acc_ref, w1_ref, b1_ref, w2_ref, b2_ref, *rest):
        if has_next:
            wu_ref, ws_ref, bs_ref, z_ref, u_ref, zs_ref = rest
        else:
            (z_ref,) = rest
        x = _silu(acc_ref[...])
        for j in range(NONSITE):
            hh = _silu(jnp.dot(x, w1_ref[j], preferred_element_type=_F32)
                       + b1_ref[j])
            x = x + jnp.dot(hh, w2_ref[j], preferred_element_type=_F32) + b2_ref[j]
        z_ref[...] = x
        if has_next:
            u_ref[...] = jnp.dot(x, wu_ref[...], preferred_element_type=_F32)
            zs_ref[...] = (jnp.dot(x, ws_ref[...], preferred_element_type=_F32)
                           + bs_ref[...])

    in_specs = [
        pl.BlockSpec((BN, DIM), lambda j: (j, 0)),
        pl.BlockSpec((NONSITE, DIM, DIM), lambda j: (0, 0, 0)),
        pl.BlockSpec((NONSITE, 1, DIM), lambda j: (0, 0, 0)),
        pl.BlockSpec((NONSITE, DIM, DIM), lambda j: (0, 0, 0)),
        pl.BlockSpec((NONSITE, 1, DIM), lambda j: (0, 0, 0)),
    ]
    out_specs = [pl.BlockSpec((BN, DIM), lambda j: (j, 0))]
    out_shape = [jax.ShapeDtypeStruct((N, DIM), _F32)]
    args = [accout, ow1, ob1, ow2, ob2]
    if has_next:
        w_u, w_self, b_self = nxt
        in_specs += [
            pl.BlockSpec((DIM, DIM), lambda j: (0, 0)),
            pl.BlockSpec((DIM, DIM), lambda j: (0, 0)),
            pl.BlockSpec((1, DIM), lambda j: (0, 0)),
        ]
        out_specs += [
            pl.BlockSpec((BN, DIM), lambda j: (j, 0)),
            pl.BlockSpec((BN, DIM), lambda j: (j, 0)),
        ]
        out_shape += [
            jax.ShapeDtypeStruct((N, DIM), _F32),
            jax.ShapeDtypeStruct((NPAD2, DIM), _F32),
        ]
        args += [w_u, w_self, b_self]

    return pl.pallas_call(
        body,
        grid=(GN,),
        in_specs=in_specs,
        out_specs=out_specs,
        out_shape=out_shape,
    )(*args)


def kernel(species, edge_src, edge_dst, distances, switch, species_table,
           Wself0, Wself1, bself, mW1_0, mW1_1, mb1, mW2, mb2,
           oW1, ob1, oW2, ob2):
    species2 = species.reshape(N, 1).astype(jnp.int32)
    pad = E_PAD - E
    dist2 = jnp.pad(distances, (0, pad), constant_values=1.0).reshape(E_PAD, 1)
    sw2 = jnp.pad(switch, (0, pad), constant_values=0.0).reshape(E_PAD, 1)
    edge_srcp = jnp.pad(edge_src.astype(jnp.int32), (0, pad),
                        constant_values=N)
    edge_dstp = jnp.pad(edge_dst.astype(jnp.int32), (0, pad),
                        constant_values=0)
    mus2 = jnp.linspace(1.0 / CUTOFF, 1.0, RDIM).reshape(1, RDIM)

    ob1r = ob1.reshape(NLAYERS, NONSITE, 1, DIM)
    ob2r = ob2.reshape(NLAYERS, NONSITE, 1, DIM)

    u, zself = _node_pre0(species2, species_table, mW1_0[:SDIM], Wself0,
                          bself[0].reshape(1, DIM))
    g = _sc_gather(u, edge_dstp)
    m = _edge_mlp(g, dist2, sw2, mus2, mW1_0[SDIM:],
                  mb1[0].reshape(1, DIM), mW2[0], mb2[0].reshape(1, DIM))
    acc = _sc_scatter(m, edge_srcp, zself)
    z0, u, zself = _node_post(acc, oW1[0], ob1r[0], oW2[0], ob2r[0],
                              (mW1_1[:DIM], Wself1, bself[1].reshape(1, DIM)))

    g = _sc_gather(u, edge_dstp)
    m = _edge_mlp(g, dist2, sw2, mus2, mW1_1[DIM:],
                  mb1[1].reshape(1, DIM), mW2[1], mb2[1].reshape(1, DIM))
    acc = _sc_scatter(m, edge_srcp, zself)
    (z1,) = _node_post(acc, oW1[1], ob1r[1], oW2[1], ob2r[1], None)

    return jnp.stack([z0, z1], axis=1)


# speedup vs baseline: 1.4357x; 1.0003x over previous
import functools

import jax
import jax.numpy as jnp
from jax import lax
from jax.experimental import pallas as pl
from jax.experimental.pallas import tpu as pltpu
from jax.experimental.pallas import tpu_sc as plsc

N = 50000
E = 800000
DIM = 80
HD = DIM // 2
RDIM = 20
SDIM = 16
NSPEC = 50
CUTOFF = 5.0
NLAYERS = 2
NONSITE = 3
ETA = (RDIM / (1.0 - 1.0 / CUTOFF)) ** 2

NTILE = 16
NWORK = 32

CH = 128
E_PAD = 802816
GCH_W = E_PAD // CH // NWORK
GGRP_W = GCH_W // 4
MEGA = 512
NMEGA_T = E_PAD // MEGA // NTILE

PHROWS = 25088
ROWS_PER_TILE = PHROWS // NTILE
NPAD2 = 2 * PHROWS
DUMMY = PHROWS

BN = 1000
GN = N // BN
BE = 6272
GE = E_PAD // BE

_F32 = jnp.float32


def _silu(x):
    return x * (1.0 / (1.0 + jnp.exp(-x)))


def _node_pre0(species2, table, w_u, w_self, b_self):
    def body(sp_ref, tab_ref, wu_ref, ws_ref, bs_ref, u_ref, zs_ref):
        sp = sp_ref[...]
        iota = lax.broadcasted_iota(jnp.int32, (BN, NSPEC), 1)
        oh = (iota == sp).astype(_F32)
        zi = jnp.dot(oh, tab_ref[...], preferred_element_type=_F32)
        u_ref[...] = jnp.dot(zi, wu_ref[...], preferred_element_type=_F32)
        zs_ref[...] = (jnp.dot(zi, ws_ref[...], preferred_element_type=_F32)
                       + bs_ref[...])

    return pl.pallas_call(
        body,
        grid=(GN,),
        in_specs=[
            pl.BlockSpec((BN, 1), lambda j: (j, 0)),
            pl.BlockSpec((NSPEC, SDIM), lambda j: (0, 0)),
            pl.BlockSpec((SDIM, DIM), lambda j: (0, 0)),
            pl.BlockSpec((SDIM, DIM), lambda j: (0, 0)),
            pl.BlockSpec((1, DIM), lambda j: (0, 0)),
        ],
        out_specs=[
            pl.BlockSpec((BN, DIM), lambda j: (j, 0)),
            pl.BlockSpec((BN, DIM), lambda j: (j, 0)),
        ],
        out_shape=[
            jax.ShapeDtypeStruct((N, DIM), _F32),
            jax.ShapeDtypeStruct((NPAD2, DIM), _F32),
        ],
    )(species2, table, w_u, w_self, b_self)


def _sc_gather(u, edge_dst):
    mesh = plsc.VectorSubcoreMesh(core_axis_name="c", subcore_axis_name="s")

    scratch = ([pltpu.VMEM((CH,), jnp.int32) for _ in range(8)]
               + [pltpu.VMEM((CH, DIM), _F32) for _ in range(8)]
               + [pltpu.SemaphoreType.DMA] * 3)

    @functools.partial(
        pl.kernel,
        out_type=jax.ShapeDtypeStruct((E_PAD, DIM), _F32),
        mesh=mesh,
        compiler_params=pltpu.CompilerParams(use_tc_tiling_on_sc=False),
        scratch_types=scratch,
    )
    def k(u_hbm, dst_hbm, g_hbm, *sc):
        idxs = sc[0:8]
        rows = sc[8:16]
        sem_i, sem_g, sem_o = sc[16:19]
        w = lax.axis_index("s") * 2 + lax.axis_index("c")
        base = w * (GCH_W * CH)

        def _drain_out(lo):
            for j in range(4):
                pltpu.make_async_copy(rows[(lo ^ 4) + j].at[...],
                                      g_hbm.at[pl.ds(base, CH)],
                                      sem_o).wait()

        def one(g_idx, lo, first):
            off0 = base + g_idx * (4 * CH)
            for j in range(4):
                pltpu.async_copy(dst_hbm.at[pl.ds(off0 + j * CH, CH)],
                                 idxs[lo + j].at[...], sem_i)
            for j in range(4):
                pltpu.make_async_copy(dst_hbm.at[pl.ds(base, CH)],
                                      idxs[lo + j].at[...], sem_i).wait()
            for j in range(4):
                pltpu.async_copy(u_hbm.at[idxs[lo + j]],
                                 rows[lo + j].at[...], sem_g)
            if first is None:
                _drain_out(lo)
            else:
                @pl.when(first)
                def _():
                    _drain_out(lo)
            for j in range(4):
                pltpu.make_async_copy(u_hbm.at[pl.ds(base, CH)],
                                      rows[lo + j].at[...], sem_g).wait()
            for j in range(4):
                pltpu.async_copy(rows[lo + j].at[...],
                                 g_hbm.at[pl.ds(off0 + j * CH, CH)], sem_o)

        def grp2(i, carry):
            g0 = i * 2
            one(g0, 0, g0 > 0)
            one(g0 + 1, 4, None)
            return carry

        lax.fori_loop(0, GGRP_W // 2, grp2, 0)
        one(GGRP_W - 1, 0, None)
        for j in range(4):
            pltpu.make_async_copy(rows[j].at[...],
                                  g_hbm.at[pl.ds(base, CH)], sem_o).wait()

    return k(u, edge_dst)


def _edge_mlp(g, dist2, sw2, mus2, w_s, b1, w2, b2):
    def body(g_ref, d_ref, sw_ref, mus_ref, ws_ref, b1_ref, w2_ref, b2_ref,
             m_ref):
        rinv = 1.0 / d_ref[...]
        s = jnp.exp(-ETA * (rinv - mus_ref[...]) ** 2)
        pre = (g_ref[...]
               + jnp.dot(s, ws_ref[...], preferred_element_type=_F32)
               + b1_ref[...])
        h = _silu(pre)
        mm = jnp.dot(h, w2_ref[...], preferred_element_type=_F32) + b2_ref[...]
        m_ref[...] = mm * sw_ref[...]

    return pl.pallas_call(
        body,
        grid=(GE,),
        in_specs=[
            pl.BlockSpec((BE, DIM), lambda j: (j, 0)),
            pl.BlockSpec((BE, 1), lambda j: (j, 0)),
            pl.BlockSpec((BE, 1), lambda j: (j, 0)),
            pl.BlockSpec((1, RDIM), lambda j: (0, 0)),
            pl.BlockSpec((RDIM, DIM), lambda j: (0, 0)),
            pl.BlockSpec((1, DIM), lambda j: (0, 0)),
            pl.BlockSpec((DIM, DIM), lambda j: (0, 0)),
            pl.BlockSpec((1, DIM), lambda j: (0, 0)),
        ],
        out_specs=pl.BlockSpec((BE, DIM), lambda j: (j, 0)),
        out_shape=jax.ShapeDtypeStruct((E_PAD, DIM), _F32),
    )(g, dist2, sw2, mus2, w_s, b1, w2, b2)


def _sc_scatter(m, edge_src, zself_pad):
    mesh = plsc.VectorSubcoreMesh(core_axis_name="c", subcore_axis_name="s")

    scratch = ([pltpu.VMEM((MEGA,), jnp.int32) for _ in range(2)]
               + [pltpu.VMEM((4, CH), jnp.int32) for _ in range(2)]
               + [pltpu.VMEM((MEGA, HD), _F32) for _ in range(2)]
               + [pltpu.VMEM_SHARED((PHROWS + 8, HD), _F32)]
               + [pltpu.SemaphoreType.DMA] * 2)

    @functools.partial(
        pl.kernel,
        out_type=jax.ShapeDtypeStruct((NPAD2, DIM), _F32),
        mesh=mesh,
        compiler_params=pltpu.CompilerParams(use_tc_tiling_on_sc=False),
        scratch_types=scratch,
    )
    def k(m_hbm, src_hbm, zs_hbm, out_hbm, *sc):
        srcs = sc[0:2]
        idx2 = sc[2:4]
        rows = sc[4:6]
        acc = sc[6]
        sem_in, sem_a = sc[7:9]
        c = lax.axis_index("c")
        s = lax.axis_index("s")
        r0 = s * ROWS_PER_TILE
        col0 = c * HD
        tbase = s * (NMEGA_T * MEGA)

        def fire_loads(mg, b):
            off = tbase + mg * MEGA
            pltpu.async_copy(src_hbm.at[pl.ds(off, MEGA)],
                             srcs[b].at[...], sem_in)
            pltpu.async_copy(m_hbm.at[pl.ds(off, MEGA), pl.ds(col0, HD)],
                             rows[b].at[...], sem_in)

        def drain_loads(b):
            pltpu.make_async_copy(src_hbm.at[pl.ds(tbase, MEGA)],
                                  srcs[b].at[...], sem_in).wait()
            pltpu.make_async_copy(
                m_hbm.at[pl.ds(tbase, MEGA), pl.ds(col0, HD)],
                rows[b].at[...], sem_in).wait()

        def drain_scat(b):
            for j in range(4):
                pltpu.make_async_copy(rows[b].at[pl.ds(j * CH, CH)],
                                      acc.at[pl.ds(0, CH)], sem_a).wait()

        for phase in range(2):
            prow = phase * PHROWS
            pltpu.sync_copy(
                zs_hbm.at[pl.ds(prow + r0, ROWS_PER_TILE), pl.ds(col0, HD)],
                acc.at[pl.ds(r0, ROWS_PER_TILE)])
            plsc.subcore_barrier()

            fire_loads(0, 0)

            def step(mg, b):
                drain_loads(b)
                for j in range(4):
                    for v in range(CH // 16):
                        sl = pl.ds(j * CH + v * 16, 16)
                        loc = srcs[b][sl] - prow
                        ok = (loc >= 0) & (loc < PHROWS)
                        idx2[b][j, pl.ds(v * 16, 16)] = jnp.where(
                            ok, loc, DUMMY)
                for j in range(4):
                    pltpu.async_copy(rows[b].at[pl.ds(j * CH, CH)],
                                     acc.at[idx2[b].at[j]], sem_a,
                                     add=True)

                @pl.when(mg > 0)
                def _():
                    drain_scat(1 - b)

                @pl.when(mg + 1 < NMEGA_T)
                def _():
                    fire_loads(mg + 1, 1 - b)

            def pair(i, carry):
                step(i * 2, 0)
                step(i * 2 + 1, 1)
                return carry

            lax.fori_loop(0, NMEGA_T // 2, pair, 0)
            drain_scat(1)
            plsc.subcore_barrier()
            pltpu.sync_copy(
                acc.at[pl.ds(r0, ROWS_PER_TILE)],
                out_hbm.at[pl.ds(prow + r0, ROWS_PER_TILE), pl.ds(col0, HD)])
            plsc.subcore_barrier()

    return k(m, edge_src, zself_pad)


def _node_post(accout, ow1, ob1, ow2, ob2, nxt):
    has_next = nxt is not None

    def body(acc_ref, w1_ref, b1_ref, w2_ref, b2_ref, *rest):
        if has_next:
            wu_ref, ws_ref, bs_ref, z_ref, u_ref, zs_ref = rest
        else:
            (z_ref,) = rest
        x = _silu(acc_ref[...])
        for j in range(NONSITE):
            hh = _silu(jnp.dot(x, w1_ref[j], preferred_element_type=_F32)
                       + b1_ref[j])
            x = x + jnp.dot(hh, w2_ref[j], preferred_element_type=_F32) + b2_ref[j]
        z_ref[...] = x
        if has_next:
            u_ref[...] = jnp.dot(x, wu_ref[...], preferred_element_type=_F32)
            zs_ref[...] = (jnp.dot(x, ws_ref[...], preferred_element_type=_F32)
                           + bs_ref[...])

    in_specs = [
        pl.BlockSpec((BN, DIM), lambda j: (j, 0)),
        pl.BlockSpec((NONSITE, DIM, DIM), lambda j: (0, 0, 0)),
        pl.BlockSpec((NONSITE, 1, DIM), lambda j: (0, 0, 0)),
        pl.BlockSpec((NONSITE, DIM, DIM), lambda j: (0, 0, 0)),
        pl.BlockSpec((NONSITE, 1, DIM), lambda j: (0, 0, 0)),
    ]
    out_specs = [pl.BlockSpec((BN, DIM), lambda j: (j, 0))]
    out_shape = [jax.ShapeDtypeStruct((N, DIM), _F32)]
    args = [accout, ow1, ob1, ow2, ob2]
    if has_next:
        w_u, w_self, b_self = nxt
        in_specs += [
            pl.BlockSpec((DIM, DIM), lambda j: (0, 0)),
            pl.BlockSpec((DIM, DIM), lambda j: (0, 0)),
            pl.BlockSpec((1, DIM), lambda j: (0, 0)),
        ]
        out_specs += [
            pl.BlockSpec((BN, DIM), lambda j: (j, 0)),
            pl.BlockSpec((BN, DIM), lambda j: (j, 0)),
        ]
        out_shape += [
            jax.ShapeDtypeStruct((N, DIM), _F32),
            jax.ShapeDtypeStruct((NPAD2, DIM), _F32),
        ]
        args += [w_u, w_self, b_self]

    return pl.pallas_call(
        body,
        grid=(GN,),
        in_specs=in_specs,
        out_specs=out_specs,
        out_shape=out_shape,
    )(*args)


def kernel(species, edge_src, edge_dst, distances, switch, species_table,
           Wself0, Wself1, bself, mW1_0, mW1_1, mb1, mW2, mb2,
           oW1, ob1, oW2, ob2):
    species2 = species.reshape(N, 1).astype(jnp.int32)
    pad = E_PAD - E
    dist2 = jnp.pad(distances, (0, pad), constant_values=1.0).reshape(E_PAD, 1)
    sw2 = jnp.pad(switch, (0, pad), constant_values=0.0).reshape(E_PAD, 1)
    edge_srcp = jnp.pad(edge_src.astype(jnp.int32), (0, pad),
                        constant_values=N)
    edge_dstp = jnp.pad(edge_dst.astype(jnp.int32), (0, pad),
                        constant_values=0)
    mus2 = jnp.linspace(1.0 / CUTOFF, 1.0, RDIM).reshape(1, RDIM)

    ob1r = ob1.reshape(NLAYERS, NONSITE, 1, DIM)
    ob2r = ob2.reshape(NLAYERS, NONSITE, 1, DIM)

    u, zself = _node_pre0(species2, species_table, mW1_0[:SDIM], Wself0,
                          bself[0].reshape(1, DIM))
    g = _sc_gather(u, edge_dstp)
    m = _edge_mlp(g, dist2, sw2, mus2, mW1_0[SDIM:],
                  mb1[0].reshape(1, DIM), mW2[0], mb2[0].reshape(1, DIM))
    acc = _sc_scatter(m, edge_srcp, zself)
    z0, u, zself = _node_post(acc, oW1[0], ob1r[0], oW2[0], ob2r[0],
                              (mW1_1[:DIM], Wself1, bself[1].reshape(1, DIM)))

    g = _sc_gather(u, edge_dstp)
    m = _edge_mlp(g, dist2, sw2, mus2, mW1_1[DIM:],
                  mb1[1].reshape(1, DIM), mW2[1], mb2[1].reshape(1, DIM))
    acc = _sc_scatter(m, edge_srcp, zself)
    (z1,) = _node_post(acc, oW1[1], ob1r[1], oW2[1], ob2r[1], None)

    return jnp.stack([z0, z1], axis=1)
